# baseline (speedup 1.0000x reference)
import functools

import jax
import jax.numpy as jnp
from jax import lax
from jax.experimental import pallas as pl
from jax.experimental.pallas import tpu as pltpu
from jax.experimental.pallas import tpu_sc as plsc

N = 10000
D = 128
E = 320000
NC = 2
NS = 16
NT = NC * NS
EPT = E // NT
CHUNK = 128
NCHUNK = 80
PADE = NCHUNK * CHUNK
DUMMY = N
NP = N + NS
RPT = NP // NS
PADN = 640 * NS

_mesh = plsc.VectorSubcoreMesh(core_axis_name="c", subcore_axis_name="s")


def _deg_body(col3, row3, val3, eq3, zer_h,
              degp, selfp, degq, selfq,
              colbuf, rowbuf, valbuf, eqbuf, zb, deg_s, self_s):
    cc = lax.axis_index("c")
    ss = lax.axis_index("s")
    tile = cc * NS + ss
    pltpu.sync_copy(zer_h, zb)
    pltpu.sync_copy(zb, deg_s.at[pl.ds(ss * 640, 640)])
    pltpu.sync_copy(zb, self_s.at[pl.ds(ss * 640, 640)])
    pltpu.sync_copy(col3.at[tile], colbuf)
    pltpu.sync_copy(row3.at[tile], rowbuf)
    pltpu.sync_copy(val3.at[tile], valbuf)
    pltpu.sync_copy(eq3.at[tile], eqbuf)
    plsc.subcore_barrier()

    def body(j, _):
        pltpu.sync_copy(valbuf.at[j], deg_s.at[colbuf.at[j]], add=True)
        pltpu.sync_copy(eqbuf.at[j], self_s.at[rowbuf.at[j]], add=True)
        return _

    lax.fori_loop(0, NCHUNK, body, None)
    plsc.subcore_barrier()

    @pl.when(cc == 0)
    def _():
        pltpu.sync_copy(deg_s.at[pl.ds(ss * 640, 640)],
                        degp.at[pl.ds(ss * 640, 640)])
        pltpu.sync_copy(self_s.at[pl.ds(ss * 640, 640)],
                        selfp.at[pl.ds(ss * 640, 640)])

    @pl.when(cc == 1)
    def _():
        pltpu.sync_copy(deg_s.at[pl.ds(ss * 640, 640)],
                        degq.at[pl.ds(ss * 640, 640)])
        pltpu.sync_copy(self_s.at[pl.ds(ss * 640, 640)],
                        selfq.at[pl.ds(ss * 640, 640)])


_deg_kernel = pl.kernel(
    _deg_body,
    out_type=(jax.ShapeDtypeStruct((PADN,), jnp.float32),
              jax.ShapeDtypeStruct((PADN,), jnp.float32),
              jax.ShapeDtypeStruct((PADN,), jnp.float32),
              jax.ShapeDtypeStruct((PADN,), jnp.float32)),
    mesh=_mesh,
    scratch_types=[
        pltpu.VMEM((NCHUNK, CHUNK), jnp.int32),
        pltpu.VMEM((NCHUNK, CHUNK), jnp.int32),
        pltpu.VMEM((NCHUNK, CHUNK), jnp.float32),
        pltpu.VMEM((NCHUNK, CHUNK), jnp.float32),
        pltpu.VMEM((640,), jnp.float32),
        pltpu.VMEM_SHARED((PADN,), jnp.float32),
        pltpu.VMEM_SHARED((PADN,), jnp.float32),
    ],
)


def _spmm_body(xp, pk3, zrows,
               tp,
               pkbuf, rcol, rrow, bufa, bufb, ys, gsa, gsb, ssa, ssb):
    cc = lax.axis_index("c")
    ss = lax.axis_index("s")
    tile = cc * NS + ss
    pltpu.sync_copy(zrows, bufa)
    base = ss * RPT
    for i in range(RPT // CHUNK):
        pltpu.sync_copy(bufa, ys.at[pl.ds(base + i * CHUNK, CHUNK)])
    rem = RPT % CHUNK
    if rem:
        pltpu.sync_copy(bufa.at[pl.ds(0, rem)],
                        ys.at[pl.ds(base + RPT - rem, rem)])
    pltpu.sync_copy(pk3.at[tile], pkbuf)
    plsc.subcore_barrier()

    def unpack(j, p):
        src = pkbuf.at[j]
        for i in range(CHUNK // 16):
            pk = src[pl.ds(i * 16, 16)]
            rcol[p, pl.ds(i * 16, 16)] = jnp.bitwise_and(pk, 16383)
            rrow[p, pl.ds(i * 16, 16)] = lax.shift_right_logical(pk, 14)

    def wait_gather(buf, sem):
        pltpu.make_async_copy(xp.at[rcol.at[0]], buf, sem).wait()

    def wait_scatter(buf, sem):
        pltpu.make_async_copy(buf, ys.at[rrow.at[0]], sem).wait()

    def body(j, _):
        unpack(j, 0)
        pltpu.sync_copy(xp.at[rcol.at[0]], bufa)
        pltpu.sync_copy(bufa, ys.at[rrow.at[0]], add=True)
        return _

    lax.fori_loop(0, NCHUNK, body, None)
    plsc.subcore_barrier()
    pltpu.sync_copy(ys.at[pl.ds(base, RPT)], tp.at[cc, ss])


_spmm_kernel = pl.kernel(
    _spmm_body,
    out_type=jax.ShapeDtypeStruct((NC, NS, RPT, D), jnp.float32),
    mesh=_mesh,
    scratch_types=[
        pltpu.VMEM((NCHUNK, CHUNK), jnp.int32),
        pltpu.VMEM((2, CHUNK), jnp.int32),
        pltpu.VMEM((2, CHUNK), jnp.int32),
        pltpu.VMEM((CHUNK, D), jnp.float32),
        pltpu.VMEM((CHUNK, D), jnp.float32),
        pltpu.VMEM_SHARED((NP, D), jnp.float32),
        pltpu.SemaphoreType.DMA,
        pltpu.SemaphoreType.DMA,
        pltpu.SemaphoreType.DMA,
        pltpu.SemaphoreType.DMA,
    ],
)


BR = 1000


def _dense_body(x_ref, t_ref, c_ref, d_ref, phi_ref, w_ref, b_ref,
                out_ref, outp_ref, *, relu):
    t = t_ref[0] + t_ref[1]
    phi = phi_ref[...]
    u = x_ref[...] * (1.0 - c_ref[...] * phi) + t * (d_ref[...] * phi)
    h = jnp.dot(u, w_ref[...], preferred_element_type=jnp.float32) + b_ref[...]
    if relu:
        h = jnp.maximum(h, 0.0)
    out_ref[...] = h
    if outp_ref is not None:
        outp_ref[...] = h * d_ref[...]


def _mid_body(x_ref, t_ref, c_ref, d_ref, phi_ref, out_ref, outp_ref):
    t = t_ref[0] + t_ref[1]
    phi = phi_ref[...]
    h = x_ref[...] * (1.0 - c_ref[...] * phi) + t * (d_ref[...] * phi)
    out_ref[...] = h
    outp_ref[...] = h * d_ref[...]


_bs_x = pl.BlockSpec((BR, D), lambda i: (i, 0))
_bs_t = pl.BlockSpec((NC, BR, D), lambda i: (0, i, 0))
_bs_n1 = pl.BlockSpec((BR, 1), lambda i: (i, 0))
_bs_row = pl.BlockSpec((1, D), lambda i: (0, 0))
_bs_w = pl.BlockSpec((D, D), lambda i: (0, 0))

_dense1 = pl.pallas_call(
    functools.partial(_dense_body, relu=True),
    grid=(N // BR,),
    in_specs=[_bs_x, _bs_t, _bs_n1, _bs_n1, _bs_row, _bs_w, _bs_row],
    out_specs=(_bs_x, _bs_x),
    out_shape=(jax.ShapeDtypeStruct((N, D), jnp.float32),
               jax.ShapeDtypeStruct((N, D), jnp.float32)),
)


def _dense_final_body(x_ref, t_ref, c_ref, d_ref, phi_ref, w_ref, b_ref,
                      out_ref):
    _dense_body(x_ref, t_ref, c_ref, d_ref, phi_ref, w_ref, b_ref,
                out_ref, None, relu=False)


_dense2 = pl.pallas_call(
    _dense_final_body,
    grid=(N // BR,),
    in_specs=[_bs_x, _bs_t, _bs_n1, _bs_n1, _bs_row, _bs_w, _bs_row],
    out_specs=_bs_x,
    out_shape=jax.ShapeDtypeStruct((N, D), jnp.float32),
)

_mid = pl.pallas_call(
    _mid_body,
    grid=(N // BR,),
    in_specs=[_bs_x, _bs_t, _bs_n1, _bs_n1, _bs_row],
    out_specs=(_bs_x, _bs_x),
    out_shape=(jax.ShapeDtypeStruct((N, D), jnp.float32),
               jax.ShapeDtypeStruct((N, D), jnp.float32)),
)


def kernel(node_feat, edge_index, phi1, W1, b1, phi_hidden, phi2, W2, b2):
    row_t = edge_index[0].reshape(NT, EPT)
    col_t = edge_index[1].reshape(NT, EPT)
    npad = PADE - EPT
    row_p = jnp.concatenate(
        [row_t, jnp.full((NT, npad), DUMMY, jnp.int32)], axis=1
    ).reshape(NT, NCHUNK, CHUNK)
    col_p = jnp.concatenate(
        [col_t, jnp.zeros((NT, npad), jnp.int32)], axis=1
    ).reshape(NT, NCHUNK, CHUNK)
    val3 = jnp.concatenate(
        [jnp.ones((NT, EPT), jnp.float32), jnp.zeros((NT, npad), jnp.float32)],
        axis=1).reshape(NT, NCHUNK, CHUNK)
    eq3 = (row_p == col_p).astype(jnp.float32)
    pk3 = jnp.bitwise_or(lax.shift_left(row_p, 14), col_p)
    zer_h = jnp.zeros((640,), jnp.float32)
    zrows = jnp.zeros((CHUNK, D), jnp.float32)

    degp, selfp, degq, selfq = _deg_kernel(col_p, row_p, val3, eq3, zer_h)
    deg = degp[:N] + degq[:N] + 1.0
    selfcnt = selfp[:N] + selfq[:N]
    dvec = lax.rsqrt(deg)
    cvec = (deg - 1.0) / deg + selfcnt
    c2 = cvec[:, None]
    d2 = dvec[:, None]

    x0 = node_feat
    x0p = x0 * d2

    def spmm_t(xp):
        return _spmm_kernel(xp, pk3, zrows).reshape(NC, NP, D)

    t0 = spmm_t(x0p)
    x1, x1p = _dense1(x0, t0, c2, d2, phi1[None, :], W1, b1[None, :])
    t1 = spmm_t(x1p)
    x2, x2p = _mid(x1, t1, c2, d2, phi_hidden[0][None, :])
    t2 = spmm_t(x2p)
    x3, x3p = _mid(x2, t2, c2, d2, phi_hidden[1][None, :])
    t3 = spmm_t(x3p)
    out = _dense2(x3, t3, c2, d2, phi2[None, :], W2, b2[None, :])
    return out


# speedup vs baseline: 3.2610x; 3.2610x over previous
import functools

import jax
import jax.numpy as jnp
from jax import lax
from jax.experimental import pallas as pl
from jax.experimental.pallas import tpu as pltpu
from jax.experimental.pallas import tpu_sc as plsc

N = 10000
D = 128
E = 320000
NC = 2
NS = 16
NT = NC * NS
EPT = E // NT
CHUNK = 125
NCHUNK = 80
HALF = NCHUNK // 2
NP = N
RPT = NP // NS
PADN = 640 * NS

_mesh = plsc.VectorSubcoreMesh(core_axis_name="c", subcore_axis_name="s")


def _deg_body(col3, row3, val3, eq3, zer_h,
              degp, selfp, degq, selfq,
              colbuf, rowbuf, valbuf, eqbuf, zb, deg_s, self_s):
    cc = lax.axis_index("c")
    ss = lax.axis_index("s")
    tile = cc * NS + ss
    pltpu.sync_copy(zer_h, zb)
    pltpu.sync_copy(zb, deg_s.at[pl.ds(ss * 640, 640)])
    pltpu.sync_copy(zb, self_s.at[pl.ds(ss * 640, 640)])
    pltpu.sync_copy(col3.at[tile], colbuf)
    pltpu.sync_copy(row3.at[tile], rowbuf)
    pltpu.sync_copy(val3.at[tile], valbuf)
    pltpu.sync_copy(eq3.at[tile], eqbuf)
    plsc.subcore_barrier()

    def body(j, _):
        pltpu.sync_copy(valbuf.at[j], deg_s.at[colbuf.at[j]], add=True)
        pltpu.sync_copy(eqbuf.at[j], self_s.at[rowbuf.at[j]], add=True)
        return _

    lax.fori_loop(0, NCHUNK, body, None)
    plsc.subcore_barrier()

    @pl.when(cc == 0)
    def _():
        pltpu.sync_copy(deg_s.at[pl.ds(ss * 640, 640)],
                        degp.at[pl.ds(ss * 640, 640)])
        pltpu.sync_copy(self_s.at[pl.ds(ss * 640, 640)],
                        selfp.at[pl.ds(ss * 640, 640)])

    @pl.when(cc == 1)
    def _():
        pltpu.sync_copy(deg_s.at[pl.ds(ss * 640, 640)],
                        degq.at[pl.ds(ss * 640, 640)])
        pltpu.sync_copy(self_s.at[pl.ds(ss * 640, 640)],
                        selfq.at[pl.ds(ss * 640, 640)])


_deg_kernel = pl.kernel(
    _deg_body,
    out_type=(jax.ShapeDtypeStruct((PADN,), jnp.float32),
              jax.ShapeDtypeStruct((PADN,), jnp.float32),
              jax.ShapeDtypeStruct((PADN,), jnp.float32),
              jax.ShapeDtypeStruct((PADN,), jnp.float32)),
    mesh=_mesh,
    scratch_types=[
        pltpu.VMEM((NCHUNK, CHUNK), jnp.int32),
        pltpu.VMEM((NCHUNK, CHUNK), jnp.int32),
        pltpu.VMEM((NCHUNK, CHUNK), jnp.float32),
        pltpu.VMEM((NCHUNK, CHUNK), jnp.float32),
        pltpu.VMEM((640,), jnp.float32),
        pltpu.VMEM_SHARED((PADN,), jnp.float32),
        pltpu.VMEM_SHARED((PADN,), jnp.float32),
    ],
)


def _spmm_body(xp, col3, row3, zrows,
               tp,
               colbuf, rowbuf, bufa, bufb, ys, gsa, gsb, ssa, ssb):
    cc = lax.axis_index("c")
    ss = lax.axis_index("s")
    tile = cc * NS + ss
    pltpu.sync_copy(zrows, bufa)
    base = ss * RPT
    for i in range(RPT // CHUNK):
        pltpu.sync_copy(bufa, ys.at[pl.ds(base + i * CHUNK, CHUNK)])
    plsc.subcore_barrier()

    def wait_gather(buf, sem):
        pltpu.make_async_copy(xp.at[colbuf.at[0]], buf, sem).wait()

    def wait_scatter(buf, sem):
        pltpu.make_async_copy(buf, ys.at[rowbuf.at[0]], sem).wait()

    for p in range(NCHUNK // HALF):
        pltpu.sync_copy(col3.at[tile, pl.ds(p * HALF, HALF)], colbuf)
        pltpu.sync_copy(row3.at[tile, pl.ds(p * HALF, HALF)], rowbuf)
        pltpu.async_copy(xp.at[colbuf.at[0]], bufa, gsa)

        def body(k, _):
            ja = 2 * k
            jb = 2 * k + 1

            @pl.when(k > 0)
            def _():
                wait_scatter(bufb, ssb)

            wait_gather(bufa, gsa)
            pltpu.async_copy(xp.at[colbuf.at[jb]], bufb, gsb)
            pltpu.async_copy(bufa, ys.at[rowbuf.at[ja]], ssa, add=True)

            @pl.when(k < HALF // 2 - 1)
            def _():
                wait_scatter(bufa, ssa)
                pltpu.async_copy(xp.at[colbuf.at[ja + 2]], bufa, gsa)

            wait_gather(bufb, gsb)
            pltpu.async_copy(bufb, ys.at[rowbuf.at[jb]], ssb, add=True)
            return _

        lax.fori_loop(0, HALF // 2, body, None)
        wait_scatter(bufa, ssa)
        wait_scatter(bufb, ssb)

    plsc.subcore_barrier()
    pltpu.sync_copy(ys.at[pl.ds(base, RPT)], tp.at[cc, ss])


_spmm_kernel = pl.kernel(
    _spmm_body,
    out_type=jax.ShapeDtypeStruct((NC, NS, RPT, D), jnp.float32),
    mesh=_mesh,
    scratch_types=[
        pltpu.VMEM((HALF, CHUNK), jnp.int32),
        pltpu.VMEM((HALF, CHUNK), jnp.int32),
        pltpu.VMEM((CHUNK, D), jnp.float32),
        pltpu.VMEM((CHUNK, D), jnp.float32),
        pltpu.VMEM_SHARED((NP, D), jnp.float32),
        pltpu.SemaphoreType.DMA,
        pltpu.SemaphoreType.DMA,
        pltpu.SemaphoreType.DMA,
        pltpu.SemaphoreType.DMA,
    ],
)


BR = 1000


def _dense_body(x_ref, t_ref, c_ref, d_ref, phi_ref, w_ref, b_ref,
                out_ref, outp_ref, *, relu):
    t = t_ref[0] + t_ref[1]
    phi = phi_ref[...]
    u = x_ref[...] * (1.0 - c_ref[...] * phi) + t * (d_ref[...] * phi)
    h = jnp.dot(u, w_ref[...], preferred_element_type=jnp.float32) + b_ref[...]
    if relu:
        h = jnp.maximum(h, 0.0)
    out_ref[...] = h
    if outp_ref is not None:
        outp_ref[...] = h * d_ref[...]


def _mid_body(x_ref, t_ref, c_ref, d_ref, phi_ref, out_ref, outp_ref):
    t = t_ref[0] + t_ref[1]
    phi = phi_ref[...]
    h = x_ref[...] * (1.0 - c_ref[...] * phi) + t * (d_ref[...] * phi)
    out_ref[...] = h
    outp_ref[...] = h * d_ref[...]


_bs_x = pl.BlockSpec((BR, D), lambda i: (i, 0))
_bs_t = pl.BlockSpec((NC, BR, D), lambda i: (0, i, 0))
_bs_n1 = pl.BlockSpec((BR, 1), lambda i: (i, 0))
_bs_row = pl.BlockSpec((1, D), lambda i: (0, 0))
_bs_w = pl.BlockSpec((D, D), lambda i: (0, 0))

_dense1 = pl.pallas_call(
    functools.partial(_dense_body, relu=True),
    grid=(N // BR,),
    in_specs=[_bs_x, _bs_t, _bs_n1, _bs_n1, _bs_row, _bs_w, _bs_row],
    out_specs=(_bs_x, _bs_x),
    out_shape=(jax.ShapeDtypeStruct((N, D), jnp.float32),
               jax.ShapeDtypeStruct((N, D), jnp.float32)),
)


def _dense_final_body(x_ref, t_ref, c_ref, d_ref, phi_ref, w_ref, b_ref,
                      out_ref):
    _dense_body(x_ref, t_ref, c_ref, d_ref, phi_ref, w_ref, b_ref,
                out_ref, None, relu=False)


_dense2 = pl.pallas_call(
    _dense_final_body,
    grid=(N // BR,),
    in_specs=[_bs_x, _bs_t, _bs_n1, _bs_n1, _bs_row, _bs_w, _bs_row],
    out_specs=_bs_x,
    out_shape=jax.ShapeDtypeStruct((N, D), jnp.float32),
)

_mid = pl.pallas_call(
    _mid_body,
    grid=(N // BR,),
    in_specs=[_bs_x, _bs_t, _bs_n1, _bs_n1, _bs_row],
    out_specs=(_bs_x, _bs_x),
    out_shape=(jax.ShapeDtypeStruct((N, D), jnp.float32),
               jax.ShapeDtypeStruct((N, D), jnp.float32)),
)


def kernel(node_feat, edge_index, phi1, W1, b1, phi_hidden, phi2, W2, b2):
    row3 = edge_index[0].reshape(NT, NCHUNK, CHUNK)
    col3 = edge_index[1].reshape(NT, NCHUNK, CHUNK)
    val3 = jnp.ones((NT, NCHUNK, CHUNK), jnp.float32)
    eq3 = (row3 == col3).astype(jnp.float32)
    zer_h = jnp.zeros((640,), jnp.float32)
    zrows = jnp.zeros((CHUNK, D), jnp.float32)

    degp, selfp, degq, selfq = _deg_kernel(col3, row3, val3, eq3, zer_h)
    deg = degp[:N] + degq[:N] + 1.0
    selfcnt = selfp[:N] + selfq[:N]
    dvec = lax.rsqrt(deg)
    cvec = (deg - 1.0) / deg + selfcnt
    c2 = cvec[:, None]
    d2 = dvec[:, None]

    x0 = node_feat
    x0p = x0 * d2

    def spmm_t(xp):
        return _spmm_kernel(xp, col3, row3, zrows).reshape(NC, NP, D)

    t0 = spmm_t(x0p)
    x1, x1p = _dense1(x0, t0, c2, d2, phi1[None, :], W1, b1[None, :])
    t1 = spmm_t(x1p)
    x2, x2p = _mid(x1, t1, c2, d2, phi_hidden[0][None, :])
    t2 = spmm_t(x2p)
    x3, x3p = _mid(x2, t2, c2, d2, phi_hidden[1][None, :])
    t3 = spmm_t(x3p)
    out = _dense2(x3, t3, c2, d2, phi2[None, :], W2, b2[None, :])
    return out


# speedup vs baseline: 3.2692x; 1.0025x over previous
import functools

import jax
import jax.numpy as jnp
from jax import lax
from jax.experimental import pallas as pl
from jax.experimental.pallas import tpu as pltpu
from jax.experimental.pallas import tpu_sc as plsc

N = 10000
D = 128
E = 320000
NC = 2
NS = 16
NT = NC * NS
EPT = E // NT
CHUNK = 125
NCHUNK = 80
HALF = NCHUNK // 2
NP = N
RPT = NP // NS
PADN = 640 * NS

_mesh = plsc.VectorSubcoreMesh(core_axis_name="c", subcore_axis_name="s")


def _deg_body(col3, row3, eq3, ones_h, zer_h,
              degp, selfp, degq, selfq,
              colbuf, rowbuf, onesb, eqbuf, zb, deg_s, self_s):
    cc = lax.axis_index("c")
    ss = lax.axis_index("s")
    tile = cc * NS + ss
    pltpu.sync_copy(zer_h, zb)
    pltpu.sync_copy(ones_h, onesb)
    pltpu.sync_copy(zb, deg_s.at[pl.ds(ss * 640, 640)])
    pltpu.sync_copy(zb, self_s.at[pl.ds(ss * 640, 640)])
    pltpu.sync_copy(col3.at[tile], colbuf)
    pltpu.sync_copy(row3.at[tile], rowbuf)
    pltpu.sync_copy(eq3.at[tile], eqbuf)
    plsc.subcore_barrier()

    def body(j, _):
        pltpu.sync_copy(onesb, deg_s.at[colbuf.at[j]], add=True)
        pltpu.sync_copy(eqbuf.at[j], self_s.at[rowbuf.at[j]], add=True)
        return _

    lax.fori_loop(0, NCHUNK, body, None)
    plsc.subcore_barrier()

    @pl.when(cc == 0)
    def _():
        pltpu.sync_copy(deg_s.at[pl.ds(ss * 640, 640)],
                        degp.at[pl.ds(ss * 640, 640)])
        pltpu.sync_copy(self_s.at[pl.ds(ss * 640, 640)],
                        selfp.at[pl.ds(ss * 640, 640)])

    @pl.when(cc == 1)
    def _():
        pltpu.sync_copy(deg_s.at[pl.ds(ss * 640, 640)],
                        degq.at[pl.ds(ss * 640, 640)])
        pltpu.sync_copy(self_s.at[pl.ds(ss * 640, 640)],
                        selfq.at[pl.ds(ss * 640, 640)])


_deg_kernel = pl.kernel(
    _deg_body,
    out_type=(jax.ShapeDtypeStruct((PADN,), jnp.float32),
              jax.ShapeDtypeStruct((PADN,), jnp.float32),
              jax.ShapeDtypeStruct((PADN,), jnp.float32),
              jax.ShapeDtypeStruct((PADN,), jnp.float32)),
    mesh=_mesh,
    scratch_types=[
        pltpu.VMEM((NCHUNK, CHUNK), jnp.int32),
        pltpu.VMEM((NCHUNK, CHUNK), jnp.int32),
        pltpu.VMEM((CHUNK,), jnp.float32),
        pltpu.VMEM((NCHUNK, CHUNK), jnp.float32),
        pltpu.VMEM((640,), jnp.float32),
        pltpu.VMEM_SHARED((PADN,), jnp.float32),
        pltpu.VMEM_SHARED((PADN,), jnp.float32),
    ],
)


def _spmm_body(xp, col3, row3, zrows,
               tp,
               colbuf, rowbuf, bufa, bufb, ys, gsa, gsb, ssa, ssb):
    cc = lax.axis_index("c")
    ss = lax.axis_index("s")
    tile = cc * NS + ss
    pltpu.sync_copy(zrows, bufa)
    base = ss * RPT
    for i in range(RPT // CHUNK):
        pltpu.async_copy(bufa, ys.at[pl.ds(base + i * CHUNK, CHUNK)], gsa)
    for i in range(RPT // CHUNK):
        pltpu.make_async_copy(bufa, ys.at[pl.ds(base, CHUNK)], gsa).wait()
    plsc.subcore_barrier()

    def wait_gather(buf, sem):
        pltpu.make_async_copy(xp.at[colbuf.at[0]], buf, sem).wait()

    def wait_scatter(buf, sem):
        pltpu.make_async_copy(buf, ys.at[rowbuf.at[0]], sem).wait()

    for p in range(NCHUNK // HALF):
        pltpu.sync_copy(col3.at[tile, pl.ds(p * HALF, HALF)], colbuf)
        pltpu.sync_copy(row3.at[tile, pl.ds(p * HALF, HALF)], rowbuf)
        pltpu.async_copy(xp.at[colbuf.at[0]], bufa, gsa)

        def body(k, _):
            ja = 2 * k
            jb = 2 * k + 1

            @pl.when(k > 0)
            def _():
                wait_scatter(bufb, ssb)

            wait_gather(bufa, gsa)
            pltpu.async_copy(xp.at[colbuf.at[jb]], bufb, gsb)
            pltpu.async_copy(bufa, ys.at[rowbuf.at[ja]], ssa, add=True)

            @pl.when(k < HALF // 2 - 1)
            def _():
                wait_scatter(bufa, ssa)
                pltpu.async_copy(xp.at[colbuf.at[ja + 2]], bufa, gsa)

            wait_gather(bufb, gsb)
            pltpu.async_copy(bufb, ys.at[rowbuf.at[jb]], ssb, add=True)
            return _

        lax.fori_loop(0, HALF // 2, body, None)
        wait_scatter(bufa, ssa)
        wait_scatter(bufb, ssb)

    plsc.subcore_barrier()
    pltpu.sync_copy(ys.at[pl.ds(base, RPT)], tp.at[cc, ss])


_spmm_kernel = pl.kernel(
    _spmm_body,
    out_type=jax.ShapeDtypeStruct((NC, NS, RPT, D), jnp.float32),
    mesh=_mesh,
    scratch_types=[
        pltpu.VMEM((HALF, CHUNK), jnp.int32),
        pltpu.VMEM((HALF, CHUNK), jnp.int32),
        pltpu.VMEM((CHUNK, D), jnp.float32),
        pltpu.VMEM((CHUNK, D), jnp.float32),
        pltpu.VMEM_SHARED((NP, D), jnp.float32),
        pltpu.SemaphoreType.DMA,
        pltpu.SemaphoreType.DMA,
        pltpu.SemaphoreType.DMA,
        pltpu.SemaphoreType.DMA,
    ],
)


BR = 1000


def _dense_body(x_ref, t_ref, c_ref, d_ref, phi_ref, w_ref, b_ref,
                out_ref, outp_ref, *, relu):
    t = t_ref[0] + t_ref[1]
    phi = phi_ref[...]
    u = x_ref[...] * (1.0 - c_ref[...] * phi) + t * (d_ref[...] * phi)
    h = jnp.dot(u, w_ref[...], preferred_element_type=jnp.float32) + b_ref[...]
    if relu:
        h = jnp.maximum(h, 0.0)
    out_ref[...] = h
    if outp_ref is not None:
        outp_ref[...] = h * d_ref[...]


def _mid_body(x_ref, t_ref, c_ref, d_ref, phi_ref, out_ref, outp_ref):
    t = t_ref[0] + t_ref[1]
    phi = phi_ref[...]
    h = x_ref[...] * (1.0 - c_ref[...] * phi) + t * (d_ref[...] * phi)
    out_ref[...] = h
    outp_ref[...] = h * d_ref[...]


_bs_x = pl.BlockSpec((BR, D), lambda i: (i, 0))
_bs_t = pl.BlockSpec((NC, BR, D), lambda i: (0, i, 0))
_bs_n1 = pl.BlockSpec((BR, 1), lambda i: (i, 0))
_bs_row = pl.BlockSpec((1, D), lambda i: (0, 0))
_bs_w = pl.BlockSpec((D, D), lambda i: (0, 0))

_dense1 = pl.pallas_call(
    functools.partial(_dense_body, relu=True),
    grid=(N // BR,),
    in_specs=[_bs_x, _bs_t, _bs_n1, _bs_n1, _bs_row, _bs_w, _bs_row],
    out_specs=(_bs_x, _bs_x),
    out_shape=(jax.ShapeDtypeStruct((N, D), jnp.float32),
               jax.ShapeDtypeStruct((N, D), jnp.float32)),
)


def _dense_final_body(x_ref, t_ref, c_ref, d_ref, phi_ref, w_ref, b_ref,
                      out_ref):
    _dense_body(x_ref, t_ref, c_ref, d_ref, phi_ref, w_ref, b_ref,
                out_ref, None, relu=False)


_dense2 = pl.pallas_call(
    _dense_final_body,
    grid=(N // BR,),
    in_specs=[_bs_x, _bs_t, _bs_n1, _bs_n1, _bs_row, _bs_w, _bs_row],
    out_specs=_bs_x,
    out_shape=jax.ShapeDtypeStruct((N, D), jnp.float32),
)

_mid = pl.pallas_call(
    _mid_body,
    grid=(N // BR,),
    in_specs=[_bs_x, _bs_t, _bs_n1, _bs_n1, _bs_row],
    out_specs=(_bs_x, _bs_x),
    out_shape=(jax.ShapeDtypeStruct((N, D), jnp.float32),
               jax.ShapeDtypeStruct((N, D), jnp.float32)),
)


def kernel(node_feat, edge_index, phi1, W1, b1, phi_hidden, phi2, W2, b2):
    row3 = edge_index[0].reshape(NT, NCHUNK, CHUNK)
    col3 = edge_index[1].reshape(NT, NCHUNK, CHUNK)
    eq3 = (row3 == col3).astype(jnp.float32)
    ones_h = jnp.ones((CHUNK,), jnp.float32)
    zer_h = jnp.zeros((640,), jnp.float32)
    zrows = jnp.zeros((CHUNK, D), jnp.float32)

    degp, selfp, degq, selfq = _deg_kernel(col3, row3, eq3, ones_h, zer_h)
    deg = degp[:N] + degq[:N] + 1.0
    selfcnt = selfp[:N] + selfq[:N]
    dvec = lax.rsqrt(deg)
    cvec = (deg - 1.0) / deg + selfcnt
    c2 = cvec[:, None]
    d2 = dvec[:, None]

    x0 = node_feat
    x0p = x0 * d2

    def spmm_t(xp):
        return _spmm_kernel(xp, col3, row3, zrows).reshape(NC, NP, D)

    t0 = spmm_t(x0p)
    x1, x1p = _dense1(x0, t0, c2, d2, phi1[None, :], W1, b1[None, :])
    t1 = spmm_t(x1p)
    x2, x2p = _mid(x1, t1, c2, d2, phi_hidden[0][None, :])
    t2 = spmm_t(x2p)
    x3, x3p = _mid(x2, t2, c2, d2, phi_hidden[1][None, :])
    t3 = spmm_t(x3p)
    out = _dense2(x3, t3, c2, d2, phi2[None, :], W2, b2[None, :])
    return out


# speedup vs baseline: 3.2702x; 1.0003x over previous
import functools

import jax
import jax.numpy as jnp
from jax import lax
from jax.experimental import pallas as pl
from jax.experimental.pallas import tpu as pltpu
from jax.experimental.pallas import tpu_sc as plsc

N = 10000
D = 128
E = 320000
NC = 2
NS = 16
NT = NC * NS
EPT = E // NT
CHUNK = 125
NCHUNK = 80
HALF = NCHUNK // 2
NP = N
RPT = NP // NS
PADN = 640 * NS

_mesh = plsc.VectorSubcoreMesh(core_axis_name="c", subcore_axis_name="s")


def _deg_body(col3, row3, eq3, ones_h, zer_h,
              degp, selfp, degq, selfq,
              colbuf, rowbuf, onesb, eqbuf, zb, deg_s, self_s, dsem):
    cc = lax.axis_index("c")
    ss = lax.axis_index("s")
    tile = cc * NS + ss
    pltpu.sync_copy(zer_h, zb)
    pltpu.sync_copy(ones_h, onesb)
    pltpu.sync_copy(zb, deg_s.at[pl.ds(ss * 640, 640)])
    pltpu.sync_copy(zb, self_s.at[pl.ds(ss * 640, 640)])
    pltpu.sync_copy(col3.at[tile], colbuf)
    pltpu.sync_copy(row3.at[tile], rowbuf)
    pltpu.sync_copy(eq3.at[tile], eqbuf)
    plsc.subcore_barrier()

    GRP = 8

    def body(g, _):
        for i in range(GRP):
            j = g * GRP + i
            pltpu.async_copy(onesb, deg_s.at[colbuf.at[j]], dsem, add=True)
            pltpu.async_copy(eqbuf.at[j], self_s.at[rowbuf.at[j]], dsem,
                             add=True)
        for i in range(GRP):
            pltpu.make_async_copy(onesb, deg_s.at[colbuf.at[0]], dsem).wait()
            pltpu.make_async_copy(eqbuf.at[0], self_s.at[rowbuf.at[0]],
                                  dsem).wait()
        return _

    lax.fori_loop(0, NCHUNK // GRP, body, None)
    plsc.subcore_barrier()

    @pl.when(cc == 0)
    def _():
        pltpu.sync_copy(deg_s.at[pl.ds(ss * 640, 640)],
                        degp.at[pl.ds(ss * 640, 640)])
        pltpu.sync_copy(self_s.at[pl.ds(ss * 640, 640)],
                        selfp.at[pl.ds(ss * 640, 640)])

    @pl.when(cc == 1)
    def _():
        pltpu.sync_copy(deg_s.at[pl.ds(ss * 640, 640)],
                        degq.at[pl.ds(ss * 640, 640)])
        pltpu.sync_copy(self_s.at[pl.ds(ss * 640, 640)],
                        selfq.at[pl.ds(ss * 640, 640)])


_deg_kernel = pl.kernel(
    _deg_body,
    out_type=(jax.ShapeDtypeStruct((PADN,), jnp.float32),
              jax.ShapeDtypeStruct((PADN,), jnp.float32),
              jax.ShapeDtypeStruct((PADN,), jnp.float32),
              jax.ShapeDtypeStruct((PADN,), jnp.float32)),
    mesh=_mesh,
    scratch_types=[
        pltpu.VMEM((NCHUNK, CHUNK), jnp.int32),
        pltpu.VMEM((NCHUNK, CHUNK), jnp.int32),
        pltpu.VMEM((CHUNK,), jnp.float32),
        pltpu.VMEM((NCHUNK, CHUNK), jnp.float32),
        pltpu.VMEM((640,), jnp.float32),
        pltpu.VMEM_SHARED((PADN,), jnp.float32),
        pltpu.VMEM_SHARED((PADN,), jnp.float32),
        pltpu.SemaphoreType.DMA,
    ],
)


def _spmm_body(xp, col3, row3, zrows,
               tp,
               colbuf, rowbuf, bufa, bufb, ys, gsa, gsb, ssa, ssb):
    cc = lax.axis_index("c")
    ss = lax.axis_index("s")
    tile = cc * NS + ss
    pltpu.sync_copy(zrows, bufa)
    base = ss * RPT
    for i in range(RPT // CHUNK):
        pltpu.async_copy(bufa, ys.at[pl.ds(base + i * CHUNK, CHUNK)], gsa)
    for i in range(RPT // CHUNK):
        pltpu.make_async_copy(bufa, ys.at[pl.ds(base, CHUNK)], gsa).wait()
    plsc.subcore_barrier()

    def wait_gather(buf, sem):
        pltpu.make_async_copy(xp.at[colbuf.at[0]], buf, sem).wait()

    def wait_scatter(buf, sem):
        pltpu.make_async_copy(buf, ys.at[rowbuf.at[0]], sem).wait()

    for p in range(NCHUNK // HALF):
        pltpu.sync_copy(col3.at[tile, pl.ds(p * HALF, HALF)], colbuf)
        pltpu.sync_copy(row3.at[tile, pl.ds(p * HALF, HALF)], rowbuf)
        pltpu.async_copy(xp.at[colbuf.at[0]], bufa, gsa)

        def body(k, _):
            ja = 2 * k
            jb = 2 * k + 1

            @pl.when(k > 0)
            def _():
                wait_scatter(bufb, ssb)

            wait_gather(bufa, gsa)
            pltpu.async_copy(xp.at[colbuf.at[jb]], bufb, gsb)
            pltpu.async_copy(bufa, ys.at[rowbuf.at[ja]], ssa, add=True)

            @pl.when(k < HALF // 2 - 1)
            def _():
                wait_scatter(bufa, ssa)
                pltpu.async_copy(xp.at[colbuf.at[ja + 2]], bufa, gsa)

            wait_gather(bufb, gsb)
            pltpu.async_copy(bufb, ys.at[rowbuf.at[jb]], ssb, add=True)
            return _

        lax.fori_loop(0, HALF // 2, body, None)
        wait_scatter(bufa, ssa)
        wait_scatter(bufb, ssb)

    plsc.subcore_barrier()
    pltpu.sync_copy(ys.at[pl.ds(base, RPT)], tp.at[cc, ss])


_spmm_kernel = pl.kernel(
    _spmm_body,
    out_type=jax.ShapeDtypeStruct((NC, NS, RPT, D), jnp.float32),
    mesh=_mesh,
    scratch_types=[
        pltpu.VMEM((HALF, CHUNK), jnp.int32),
        pltpu.VMEM((HALF, CHUNK), jnp.int32),
        pltpu.VMEM((CHUNK, D), jnp.float32),
        pltpu.VMEM((CHUNK, D), jnp.float32),
        pltpu.VMEM_SHARED((NP, D), jnp.float32),
        pltpu.SemaphoreType.DMA,
        pltpu.SemaphoreType.DMA,
        pltpu.SemaphoreType.DMA,
        pltpu.SemaphoreType.DMA,
    ],
)


BR = 1000


def _dense_body(x_ref, t_ref, c_ref, d_ref, phi_ref, w_ref, b_ref,
                out_ref, outp_ref, *, relu):
    t = t_ref[0] + t_ref[1]
    phi = phi_ref[...]
    u = x_ref[...] * (1.0 - c_ref[...] * phi) + t * (d_ref[...] * phi)
    h = jnp.dot(u, w_ref[...], preferred_element_type=jnp.float32) + b_ref[...]
    if relu:
        h = jnp.maximum(h, 0.0)
    out_ref[...] = h
    if outp_ref is not None:
        outp_ref[...] = h * d_ref[...]


def _mid_body(x_ref, t_ref, c_ref, d_ref, phi_ref, out_ref, outp_ref):
    t = t_ref[0] + t_ref[1]
    phi = phi_ref[...]
    h = x_ref[...] * (1.0 - c_ref[...] * phi) + t * (d_ref[...] * phi)
    out_ref[...] = h
    outp_ref[...] = h * d_ref[...]


_bs_x = pl.BlockSpec((BR, D), lambda i: (i, 0))
_bs_t = pl.BlockSpec((NC, BR, D), lambda i: (0, i, 0))
_bs_n1 = pl.BlockSpec((BR, 1), lambda i: (i, 0))
_bs_row = pl.BlockSpec((1, D), lambda i: (0, 0))
_bs_w = pl.BlockSpec((D, D), lambda i: (0, 0))

_dense1 = pl.pallas_call(
    functools.partial(_dense_body, relu=True),
    grid=(N // BR,),
    in_specs=[_bs_x, _bs_t, _bs_n1, _bs_n1, _bs_row, _bs_w, _bs_row],
    out_specs=(_bs_x, _bs_x),
    out_shape=(jax.ShapeDtypeStruct((N, D), jnp.float32),
               jax.ShapeDtypeStruct((N, D), jnp.float32)),
)


def _dense_final_body(x_ref, t_ref, c_ref, d_ref, phi_ref, w_ref, b_ref,
                      out_ref):
    _dense_body(x_ref, t_ref, c_ref, d_ref, phi_ref, w_ref, b_ref,
                out_ref, None, relu=False)


_dense2 = pl.pallas_call(
    _dense_final_body,
    grid=(N // BR,),
    in_specs=[_bs_x, _bs_t, _bs_n1, _bs_n1, _bs_row, _bs_w, _bs_row],
    out_specs=_bs_x,
    out_shape=jax.ShapeDtypeStruct((N, D), jnp.float32),
)

_mid = pl.pallas_call(
    _mid_body,
    grid=(N // BR,),
    in_specs=[_bs_x, _bs_t, _bs_n1, _bs_n1, _bs_row],
    out_specs=(_bs_x, _bs_x),
    out_shape=(jax.ShapeDtypeStruct((N, D), jnp.float32),
               jax.ShapeDtypeStruct((N, D), jnp.float32)),
)


def kernel(node_feat, edge_index, phi1, W1, b1, phi_hidden, phi2, W2, b2):
    row3 = edge_index[0].reshape(NT, NCHUNK, CHUNK)
    col3 = edge_index[1].reshape(NT, NCHUNK, CHUNK)
    eq3 = (row3 == col3).astype(jnp.float32)
    ones_h = jnp.ones((CHUNK,), jnp.float32)
    zer_h = jnp.zeros((640,), jnp.float32)
    zrows = jnp.zeros((CHUNK, D), jnp.float32)

    degp, selfp, degq, selfq = _deg_kernel(col3, row3, eq3, ones_h, zer_h)
    deg = degp[:N] + degq[:N] + 1.0
    selfcnt = selfp[:N] + selfq[:N]
    dvec = lax.rsqrt(deg)
    cvec = (deg - 1.0) / deg + selfcnt
    c2 = cvec[:, None]
    d2 = dvec[:, None]

    x0 = node_feat
    x0p = x0 * d2

    def spmm_t(xp):
        return _spmm_kernel(xp, col3, row3, zrows).reshape(NC, NP, D)

    def mid(x, t, phi):
        h = x * (1.0 - c2 * phi[None, :]) + (t[0] + t[1]) * (d2 * phi[None, :])
        return h, h * d2

    t0 = spmm_t(x0p)
    x1, x1p = _dense1(x0, t0, c2, d2, phi1[None, :], W1, b1[None, :])
    t1 = spmm_t(x1p)
    x2, x2p = mid(x1, t1, phi_hidden[0])
    t2 = spmm_t(x2p)
    x3, x3p = mid(x2, t2, phi_hidden[1])
    t3 = spmm_t(x3p)
    out = _dense2(x3, t3, c2, d2, phi2[None, :], W2, b2[None, :])
    return out


# speedup vs baseline: 3.2738x; 1.0011x over previous
import functools

import jax
import jax.numpy as jnp
from jax import lax
from jax.experimental import pallas as pl
from jax.experimental.pallas import tpu as pltpu
from jax.experimental.pallas import tpu_sc as plsc

N = 10000
D = 128
E = 320000
NC = 2
NS = 16
NT = NC * NS
EPT = E // NT
CHUNK = 125
NCHUNK = 80
HALF = NCHUNK // 2
NP = N
RPT = NP // NS
PADN = 640 * NS

_mesh = plsc.VectorSubcoreMesh(core_axis_name="c", subcore_axis_name="s")


def _deg_body(col3, row3, eq3, ones_h, zer_h,
              degp, selfp, degq, selfq,
              colbuf, rowbuf, onesb, eqbuf, zb, deg_s, self_s, dsem):
    cc = lax.axis_index("c")
    ss = lax.axis_index("s")
    tile = cc * NS + ss
    pltpu.sync_copy(zer_h, zb)
    pltpu.sync_copy(ones_h, onesb)
    pltpu.sync_copy(zb, deg_s.at[pl.ds(ss * 640, 640)])
    pltpu.sync_copy(zb, self_s.at[pl.ds(ss * 640, 640)])
    pltpu.sync_copy(col3.at[tile], colbuf)
    pltpu.sync_copy(row3.at[tile], rowbuf)
    pltpu.sync_copy(eq3.at[tile], eqbuf)
    plsc.subcore_barrier()

    GRP = 8

    def body(g, _):
        for i in range(GRP):
            j = g * GRP + i
            pltpu.async_copy(onesb, deg_s.at[colbuf.at[j]], dsem, add=True)
            pltpu.async_copy(eqbuf.at[j], self_s.at[rowbuf.at[j]], dsem,
                             add=True)
        for i in range(GRP):
            pltpu.make_async_copy(onesb, deg_s.at[colbuf.at[0]], dsem).wait()
            pltpu.make_async_copy(eqbuf.at[0], self_s.at[rowbuf.at[0]],
                                  dsem).wait()
        return _

    lax.fori_loop(0, NCHUNK // GRP, body, None)
    plsc.subcore_barrier()

    @pl.when(cc == 0)
    def _():
        pltpu.sync_copy(deg_s.at[pl.ds(ss * 640, 640)],
                        degp.at[pl.ds(ss * 640, 640)])
        pltpu.sync_copy(self_s.at[pl.ds(ss * 640, 640)],
                        selfp.at[pl.ds(ss * 640, 640)])

    @pl.when(cc == 1)
    def _():
        pltpu.sync_copy(deg_s.at[pl.ds(ss * 640, 640)],
                        degq.at[pl.ds(ss * 640, 640)])
        pltpu.sync_copy(self_s.at[pl.ds(ss * 640, 640)],
                        selfq.at[pl.ds(ss * 640, 640)])


_deg_kernel = pl.kernel(
    _deg_body,
    out_type=(jax.ShapeDtypeStruct((PADN,), jnp.float32),
              jax.ShapeDtypeStruct((PADN,), jnp.float32),
              jax.ShapeDtypeStruct((PADN,), jnp.float32),
              jax.ShapeDtypeStruct((PADN,), jnp.float32)),
    mesh=_mesh,
    scratch_types=[
        pltpu.VMEM((NCHUNK, CHUNK), jnp.int32),
        pltpu.VMEM((NCHUNK, CHUNK), jnp.int32),
        pltpu.VMEM((CHUNK,), jnp.float32),
        pltpu.VMEM((NCHUNK, CHUNK), jnp.float32),
        pltpu.VMEM((640,), jnp.float32),
        pltpu.VMEM_SHARED((PADN,), jnp.float32),
        pltpu.VMEM_SHARED((PADN,), jnp.float32),
        pltpu.SemaphoreType.DMA,
    ],
)


def _spmm_body(xp, col3, row3, zrows,
               tp,
               colbuf, rowbuf, bufa, bufb, ys, gsa, gsb, ssa, ssb):
    cc = lax.axis_index("c")
    ss = lax.axis_index("s")
    tile = cc * NS + ss
    pltpu.sync_copy(zrows, bufa)
    base = ss * RPT
    for i in range(RPT // CHUNK):
        pltpu.async_copy(bufa, ys.at[pl.ds(base + i * CHUNK, CHUNK)], gsa)
    for i in range(RPT // CHUNK):
        pltpu.make_async_copy(bufa, ys.at[pl.ds(base, CHUNK)], gsa).wait()
    plsc.subcore_barrier()

    def wait_gather(buf, sem):
        pltpu.make_async_copy(xp.at[colbuf.at[0]], buf, sem).wait()

    def wait_scatter(buf, sem):
        pltpu.make_async_copy(buf, ys.at[rowbuf.at[0]], sem).wait()

    for p in range(NCHUNK // HALF):
        pltpu.sync_copy(col3.at[tile, pl.ds(p * HALF, HALF)], colbuf)
        pltpu.sync_copy(row3.at[tile, pl.ds(p * HALF, HALF)], rowbuf)
        pltpu.async_copy(xp.at[colbuf.at[0]], bufa, gsa)

        def body(k, _):
            ja = 2 * k
            jb = 2 * k + 1

            @pl.when(k > 0)
            def _():
                wait_scatter(bufb, ssb)

            wait_gather(bufa, gsa)
            pltpu.async_copy(xp.at[colbuf.at[jb]], bufb, gsb)
            pltpu.async_copy(bufa, ys.at[rowbuf.at[ja]], ssa, add=True)

            @pl.when(k < HALF // 2 - 1)
            def _():
                wait_scatter(bufa, ssa)
                pltpu.async_copy(xp.at[colbuf.at[ja + 2]], bufa, gsa)

            wait_gather(bufb, gsb)
            pltpu.async_copy(bufb, ys.at[rowbuf.at[jb]], ssb, add=True)
            return _

        lax.fori_loop(0, HALF // 2, body, None)
        wait_scatter(bufa, ssa)
        wait_scatter(bufb, ssb)

    plsc.subcore_barrier()
    pltpu.sync_copy(ys.at[pl.ds(base, RPT)], tp.at[cc, ss])


_spmm_kernel = pl.kernel(
    _spmm_body,
    out_type=jax.ShapeDtypeStruct((NC, NS, RPT, D), jnp.float32),
    mesh=_mesh,
    scratch_types=[
        pltpu.VMEM((HALF, CHUNK), jnp.int32),
        pltpu.VMEM((HALF, CHUNK), jnp.int32),
        pltpu.VMEM((CHUNK, D), jnp.float32),
        pltpu.VMEM((CHUNK, D), jnp.float32),
        pltpu.VMEM_SHARED((NP, D), jnp.float32),
        pltpu.SemaphoreType.DMA,
        pltpu.SemaphoreType.DMA,
        pltpu.SemaphoreType.DMA,
        pltpu.SemaphoreType.DMA,
    ],
)


BR = 1000


def _dense_body(x_ref, t_ref, c_ref, d_ref, phi_ref, w_ref, b_ref,
                out_ref, outp_ref, *, relu):
    t = t_ref[0] + t_ref[1]
    phi = phi_ref[...]
    u = x_ref[...] * (1.0 - c_ref[...] * phi) + t * (d_ref[...] * phi)
    h = jnp.dot(u, w_ref[...], preferred_element_type=jnp.float32) + b_ref[...]
    if relu:
        h = jnp.maximum(h, 0.0)
    out_ref[...] = h
    if outp_ref is not None:
        outp_ref[...] = h * d_ref[...]


_bs_x = pl.BlockSpec((BR, D), lambda i: (i, 0))
_bs_t = pl.BlockSpec((NC, BR, D), lambda i: (0, i, 0))
_bs_n1 = pl.BlockSpec((BR, 1), lambda i: (i, 0))
_bs_row = pl.BlockSpec((1, D), lambda i: (0, 0))
_bs_w = pl.BlockSpec((D, D), lambda i: (0, 0))

_dense1 = pl.pallas_call(
    functools.partial(_dense_body, relu=True),
    grid=(N // BR,),
    in_specs=[_bs_x, _bs_t, _bs_n1, _bs_n1, _bs_row, _bs_w, _bs_row],
    out_specs=(_bs_x, _bs_x),
    out_shape=(jax.ShapeDtypeStruct((N, D), jnp.float32),
               jax.ShapeDtypeStruct((N, D), jnp.float32)),
)


def _dense_final_body(x_ref, t_ref, c_ref, d_ref, phi_ref, w_ref, b_ref,
                      out_ref):
    _dense_body(x_ref, t_ref, c_ref, d_ref, phi_ref, w_ref, b_ref,
                out_ref, None, relu=False)


_dense2 = pl.pallas_call(
    _dense_final_body,
    grid=(N // BR,),
    in_specs=[_bs_x, _bs_t, _bs_n1, _bs_n1, _bs_row, _bs_w, _bs_row],
    out_specs=_bs_x,
    out_shape=jax.ShapeDtypeStruct((N, D), jnp.float32),
)

def kernel(node_feat, edge_index, phi1, W1, b1, phi_hidden, phi2, W2, b2):
    row3 = edge_index[0].reshape(NT, NCHUNK, CHUNK)
    col3 = edge_index[1].reshape(NT, NCHUNK, CHUNK)
    eq3 = (row3 == col3).astype(jnp.float32)
    ones_h = jnp.ones((CHUNK,), jnp.float32)
    zer_h = jnp.zeros((640,), jnp.float32)
    zrows = jnp.zeros((CHUNK, D), jnp.float32)

    degp, selfp, degq, selfq = _deg_kernel(col3, row3, eq3, ones_h, zer_h)
    deg = degp[:N] + degq[:N] + 1.0
    selfcnt = selfp[:N] + selfq[:N]
    dvec = lax.rsqrt(deg)
    cvec = (deg - 1.0) / deg + selfcnt
    c2 = cvec[:, None]
    d2 = dvec[:, None]

    x0 = node_feat
    x0p = x0 * d2

    def spmm_t(xp):
        return _spmm_kernel(xp, col3, row3, zrows).reshape(NC, NP, D)

    def mid(x, t, phi):
        h = x * (1.0 - c2 * phi[None, :]) + (t[0] + t[1]) * (d2 * phi[None, :])
        return h, h * d2

    t0 = spmm_t(x0p)
    x1, x1p = _dense1(x0, t0, c2, d2, phi1[None, :], W1, b1[None, :])
    t1 = spmm_t(x1p)
    x2, x2p = mid(x1, t1, phi_hidden[0])
    t2 = spmm_t(x2p)
    x3, x3p = mid(x2, t2, phi_hidden[1])
    t3 = spmm_t(x3p)
    out = _dense2(x3, t3, c2, d2, phi2[None, :], W2, b2[None, :])
    return out
